# baseline (speedup 1.0000x reference)
import functools

import jax
import jax.numpy as jnp
from jax import lax
from jax.experimental import pallas as pl
from jax.experimental.pallas import tpu as pltpu

N_DOCS = 100000
EMBED_DIM = 384
BLOCK_ROWS = 2000
NUM_BLOCKS = N_DOCS // BLOCK_ROWS
KPAD = 128
ROWMAX_PAD = 64
EPS = 1e-12
NEG_INF = float("-inf")


def _rag_kernel(q_ref, corpus_ref, denom_ref, cross_ref, w1_ref, b1_ref,
                w2_ref, b2_ref, idx_out_ref, score_out_ref, sims_ref,
                rowmax_ref, *, k):
    b = pl.program_id(0)

    @pl.when(b == 0)
    def _():
        rowmax_ref[...] = jnp.full((ROWMAX_PAD, 1), NEG_INF, jnp.float32)

    cnb = (corpus_ref[...] / denom_ref[...]).astype(jnp.bfloat16)
    qb = q_ref[...].astype(jnp.bfloat16)
    acc = None
    for lo in (0, 128, 256):
        p = lax.dot_general(qb[:, lo:lo + 128], cnb[:, lo:lo + 128],
                            (((1,), (1,)), ((), ())),
                            preferred_element_type=jnp.float32)
        acc = p if acc is None else acc + p
    sims_ref[b, :] = acc[0, :]
    rowmax_ref[pl.ds(b, 1), 0:1] = jnp.max(acc).reshape(1, 1)

    @pl.when(b == NUM_BLOCKS - 1)
    def _():
        row_iota = lax.broadcasted_iota(jnp.int32, (ROWMAX_PAD, 1), 0)
        col_iota = lax.broadcasted_iota(jnp.int32, (1, BLOCK_ROWS), 1)
        kpos = lax.broadcasted_iota(jnp.int32, (1, KPAD), 1)

        def body(i, carry):
            vals, idxs = carry
            rm = rowmax_ref[...]
            m = jnp.max(rm)
            r = jnp.min(jnp.where(rm == m, row_iota, jnp.int32(ROWMAX_PAD)))
            row = sims_ref[pl.ds(r, 1), :]
            cidx = jnp.min(jnp.where(row == m, col_iota, jnp.int32(N_DOCS)))
            idx = r * BLOCK_ROWS + cidx
            new_row = jnp.where(col_iota == cidx, NEG_INF, row)
            sims_ref[pl.ds(r, 1), :] = new_row
            rowmax_ref[pl.ds(r, 1), 0:1] = jnp.max(new_row).reshape(1, 1)
            sel = kpos == i
            vals = jnp.where(sel, m, vals)
            idxs = jnp.where(sel, idx, idxs)
            return vals, idxs

        vals0 = jnp.full((1, KPAD), NEG_INF, jnp.float32)
        idxs0 = jnp.zeros((1, KPAD), jnp.int32)
        vals, idxs = lax.fori_loop(0, k, body, (vals0, idxs0))
        topk_scores = vals[:, :k]

        x = cross_ref[...]
        refined = jnp.zeros((1, k), jnp.float32)
        for j in range(32):
            h = jnp.maximum(x * w1_ref[0, j] + b1_ref[0, j], 0.0)
            h = h.astype(jnp.bfloat16).astype(jnp.float32)
            w2j = w2_ref[j, 0].astype(jnp.bfloat16).astype(jnp.float32)
            refined = refined + h * w2j
        refined = refined + b2_ref[0, 0]

        idx_out_ref[...] = idxs[:, :k]
        score_out_ref[...] = refined + 0.1 * topk_scores


def kernel(query_embed, corpus_embeds, cross_scores, W1, b1, W2, b2, k):
    k_static = cross_scores.shape[-1]
    cn = jnp.linalg.norm(corpus_embeds, ord=2, axis=-1, keepdims=True)
    cdenom = jnp.maximum(cn, EPS)
    qn = jnp.linalg.norm(query_embed, ord=2, axis=-1, keepdims=True)
    qnorm = query_embed / jnp.maximum(qn, EPS)

    kern = functools.partial(_rag_kernel, k=k_static)
    idx_out, score_out = pl.pallas_call(
        kern,
        grid=(NUM_BLOCKS,),
        in_specs=[
            pl.BlockSpec((1, EMBED_DIM), lambda b: (0, 0)),
            pl.BlockSpec((BLOCK_ROWS, EMBED_DIM), lambda b: (b, 0)),
            pl.BlockSpec((BLOCK_ROWS, 1), lambda b: (b, 0)),
            pl.BlockSpec((1, k_static), lambda b: (0, 0)),
            pl.BlockSpec((1, 32), lambda b: (0, 0)),
            pl.BlockSpec((1, 32), lambda b: (0, 0)),
            pl.BlockSpec((32, 1), lambda b: (0, 0)),
            pl.BlockSpec((1, 1), lambda b: (0, 0)),
        ],
        out_specs=[
            pl.BlockSpec((1, k_static), lambda b: (0, 0)),
            pl.BlockSpec((1, k_static), lambda b: (0, 0)),
        ],
        out_shape=[
            jax.ShapeDtypeStruct((1, k_static), jnp.int32),
            jax.ShapeDtypeStruct((1, k_static), jnp.float32),
        ],
        scratch_shapes=[pltpu.VMEM((NUM_BLOCKS, BLOCK_ROWS), jnp.float32),
                        pltpu.VMEM((ROWMAX_PAD, 1), jnp.float32)],
    )(qnorm, corpus_embeds, cdenom, cross_scores,
      W1, b1.reshape(1, 32), W2, b2.reshape(1, 1))
    return idx_out, score_out


# speedup vs baseline: 1.0162x; 1.0162x over previous
import functools

import jax
import jax.numpy as jnp
from jax import lax
from jax.experimental import pallas as pl
from jax.experimental.pallas import tpu as pltpu

N_DOCS = 100000
EMBED_DIM = 384
BLOCK_ROWS = 2000
NUM_BLOCKS = N_DOCS // BLOCK_ROWS
KPAD = 128
ROWMAX_PAD = 64
EPS = 1e-12
NEG_INF = float("-inf")


def _rag_kernel(q_ref, corpus_ref, denom_ref, cross_ref, w1_ref, b1_ref,
                w2_ref, b2_ref, idx_out_ref, score_out_ref, sims_ref,
                rowmax_ref, *, k):
    b = pl.program_id(0)

    @pl.when(b == 0)
    def _():
        rowmax_ref[...] = jnp.full((ROWMAX_PAD, 1), NEG_INF, jnp.float32)

    inv = 1.0 / denom_ref[...]
    cnb = (corpus_ref[...] * inv).astype(jnp.bfloat16)
    qb = q_ref[...].astype(jnp.bfloat16)
    acc = None
    for lo in (0, 128, 256):
        p = lax.dot_general(qb[:, lo:lo + 128], cnb[:, lo:lo + 128],
                            (((1,), (1,)), ((), ())),
                            preferred_element_type=jnp.float32)
        acc = p if acc is None else acc + p
    sims_ref[b, :] = acc[0, :]
    rowmax_ref[pl.ds(b, 1), 0:1] = jnp.max(acc).reshape(1, 1)

    @pl.when(b == NUM_BLOCKS - 1)
    def _():
        row_iota = lax.broadcasted_iota(jnp.int32, (ROWMAX_PAD, 1), 0)
        col_iota = lax.broadcasted_iota(jnp.int32, (1, BLOCK_ROWS), 1)
        kpos = lax.broadcasted_iota(jnp.int32, (1, KPAD), 1)

        def body(i, carry):
            vals, idxs = carry
            rm = rowmax_ref[...]
            m = jnp.max(rm)
            r = jnp.min(jnp.where(rm == m, row_iota, jnp.int32(ROWMAX_PAD)))
            row = sims_ref[pl.ds(r, 1), :]
            cidx = jnp.min(jnp.where(row == m, col_iota, jnp.int32(N_DOCS)))
            idx = r * BLOCK_ROWS + cidx
            new_row = jnp.where(col_iota == cidx, NEG_INF, row)
            sims_ref[pl.ds(r, 1), :] = new_row
            rowmax_ref[pl.ds(r, 1), 0:1] = jnp.max(new_row).reshape(1, 1)
            sel = kpos == i
            vals = jnp.where(sel, m, vals)
            idxs = jnp.where(sel, idx, idxs)
            return vals, idxs

        vals0 = jnp.full((1, KPAD), NEG_INF, jnp.float32)
        idxs0 = jnp.zeros((1, KPAD), jnp.int32)
        vals, idxs = lax.fori_loop(0, k, body, (vals0, idxs0))
        topk_scores = vals[:, :k]

        x = cross_ref[...]
        refined = jnp.zeros((1, k), jnp.float32)
        for j in range(32):
            h = jnp.maximum(x * w1_ref[0, j] + b1_ref[0, j], 0.0)
            h = h.astype(jnp.bfloat16).astype(jnp.float32)
            w2j = w2_ref[j, 0].astype(jnp.bfloat16).astype(jnp.float32)
            refined = refined + h * w2j
        refined = refined + b2_ref[0, 0]

        idx_out_ref[...] = idxs[:, :k]
        score_out_ref[...] = refined + 0.1 * topk_scores


def kernel(query_embed, corpus_embeds, cross_scores, W1, b1, W2, b2, k):
    k_static = cross_scores.shape[-1]
    cn = jnp.linalg.norm(corpus_embeds, ord=2, axis=-1, keepdims=True)
    cdenom = jnp.maximum(cn, EPS)
    qn = jnp.linalg.norm(query_embed, ord=2, axis=-1, keepdims=True)
    qnorm = query_embed / jnp.maximum(qn, EPS)

    kern = functools.partial(_rag_kernel, k=k_static)
    idx_out, score_out = pl.pallas_call(
        kern,
        grid=(NUM_BLOCKS,),
        in_specs=[
            pl.BlockSpec((1, EMBED_DIM), lambda b: (0, 0)),
            pl.BlockSpec((BLOCK_ROWS, EMBED_DIM), lambda b: (b, 0)),
            pl.BlockSpec((BLOCK_ROWS, 1), lambda b: (b, 0)),
            pl.BlockSpec((1, k_static), lambda b: (0, 0)),
            pl.BlockSpec((1, 32), lambda b: (0, 0)),
            pl.BlockSpec((1, 32), lambda b: (0, 0)),
            pl.BlockSpec((32, 1), lambda b: (0, 0)),
            pl.BlockSpec((1, 1), lambda b: (0, 0)),
        ],
        out_specs=[
            pl.BlockSpec((1, k_static), lambda b: (0, 0)),
            pl.BlockSpec((1, k_static), lambda b: (0, 0)),
        ],
        out_shape=[
            jax.ShapeDtypeStruct((1, k_static), jnp.int32),
            jax.ShapeDtypeStruct((1, k_static), jnp.float32),
        ],
        scratch_shapes=[pltpu.VMEM((NUM_BLOCKS, BLOCK_ROWS), jnp.float32),
                        pltpu.VMEM((ROWMAX_PAD, 1), jnp.float32)],
    )(qnorm, corpus_embeds, cdenom, cross_scores,
      W1, b1.reshape(1, 32), W2, b2.reshape(1, 1))
    return idx_out, score_out


# speedup vs baseline: 1.0668x; 1.0498x over previous
import functools

import jax
import jax.numpy as jnp
from jax import lax
from jax.experimental import pallas as pl
from jax.experimental.pallas import tpu as pltpu

N_DOCS = 100000
EMBED_DIM = 384
BLOCK_ROWS = 2000
NUM_BLOCKS = N_DOCS // BLOCK_ROWS
KPAD = 128
ROWMAX_PAD = 64
EPS = 1e-12
NEG_INF = float("-inf")


def _rag_kernel(q_ref, corpus_ref, denom_ref, cross_ref, w1_ref, b1_ref,
                w2_ref, b2_ref, idx_out_ref, score_out_ref, sims_ref, *, k):
    b = pl.program_id(0)

    inv = 1.0 / denom_ref[...]
    cnb = (corpus_ref[...] * inv).astype(jnp.bfloat16)
    qb = q_ref[...].astype(jnp.bfloat16)
    acc = None
    for lo in (0, 128, 256):
        p = lax.dot_general(qb[:, lo:lo + 128], cnb[:, lo:lo + 128],
                            (((1,), (1,)), ((), ())),
                            preferred_element_type=jnp.float32)
        acc = p if acc is None else acc + p
    sims_ref[b, :] = acc[0, :]

    @pl.when(b == NUM_BLOCKS - 1)
    def _():
        row_iota = lax.broadcasted_iota(jnp.int32, (NUM_BLOCKS, BLOCK_ROWS), 0)
        col_iota = lax.broadcasted_iota(jnp.int32, (NUM_BLOCKS, BLOCK_ROWS), 1)
        flat = row_iota * BLOCK_ROWS + col_iota
        kpos = lax.broadcasted_iota(jnp.int32, (1, KPAD), 1)

        def body(i, carry):
            vals, idxs = carry
            s = sims_ref[...]
            m = jnp.max(s)
            idx = jnp.min(jnp.where(s == m, flat, jnp.int32(N_DOCS)))
            sims_ref[...] = jnp.where(flat == idx, NEG_INF, s)
            sel = kpos == i
            vals = jnp.where(sel, m, vals)
            idxs = jnp.where(sel, idx, idxs)
            return vals, idxs

        vals0 = jnp.full((1, KPAD), NEG_INF, jnp.float32)
        idxs0 = jnp.zeros((1, KPAD), jnp.int32)
        vals, idxs = lax.fori_loop(0, k, body, (vals0, idxs0))
        topk_scores = vals[:, :k]

        x = cross_ref[...]
        refined = jnp.zeros((1, k), jnp.float32)
        for j in range(32):
            h = jnp.maximum(x * w1_ref[0, j] + b1_ref[0, j], 0.0)
            h = h.astype(jnp.bfloat16).astype(jnp.float32)
            w2j = w2_ref[j, 0].astype(jnp.bfloat16).astype(jnp.float32)
            refined = refined + h * w2j
        refined = refined + b2_ref[0, 0]

        idx_out_ref[...] = idxs[:, :k]
        score_out_ref[...] = refined + 0.1 * topk_scores


def kernel(query_embed, corpus_embeds, cross_scores, W1, b1, W2, b2, k):
    k_static = cross_scores.shape[-1]
    cn = jnp.linalg.norm(corpus_embeds, ord=2, axis=-1, keepdims=True)
    cdenom = jnp.maximum(cn, EPS)
    qn = jnp.linalg.norm(query_embed, ord=2, axis=-1, keepdims=True)
    qnorm = query_embed / jnp.maximum(qn, EPS)

    kern = functools.partial(_rag_kernel, k=k_static)
    idx_out, score_out = pl.pallas_call(
        kern,
        grid=(NUM_BLOCKS,),
        in_specs=[
            pl.BlockSpec((1, EMBED_DIM), lambda b: (0, 0)),
            pl.BlockSpec((BLOCK_ROWS, EMBED_DIM), lambda b: (b, 0)),
            pl.BlockSpec((BLOCK_ROWS, 1), lambda b: (b, 0)),
            pl.BlockSpec((1, k_static), lambda b: (0, 0)),
            pl.BlockSpec((1, 32), lambda b: (0, 0)),
            pl.BlockSpec((1, 32), lambda b: (0, 0)),
            pl.BlockSpec((32, 1), lambda b: (0, 0)),
            pl.BlockSpec((1, 1), lambda b: (0, 0)),
        ],
        out_specs=[
            pl.BlockSpec((1, k_static), lambda b: (0, 0)),
            pl.BlockSpec((1, k_static), lambda b: (0, 0)),
        ],
        out_shape=[
            jax.ShapeDtypeStruct((1, k_static), jnp.int32),
            jax.ShapeDtypeStruct((1, k_static), jnp.float32),
        ],
        scratch_shapes=[pltpu.VMEM((NUM_BLOCKS, BLOCK_ROWS), jnp.float32)],
    )(qnorm, corpus_embeds, cdenom, cross_scores,
      W1, b1.reshape(1, 32), W2, b2.reshape(1, 1))
    return idx_out, score_out
